# baseline (speedup 1.0000x reference)
import functools

import jax
import jax.numpy as jnp
from jax import lax
from jax.experimental import pallas as pl
from jax.experimental.pallas import tpu as pltpu
from jax.experimental.pallas import tpu_sc as plsc

_NUM_ENTITIES = 100000
_EMBEDDING_DIM = 64
_BATCH = 4096

_NUM_CORES = 2
_NUM_SUBCORES = 16
_NUM_WORKERS = _NUM_CORES * _NUM_SUBCORES
_B_PER_W = _BATCH // _NUM_WORKERS
_LANES = 16

_mesh = plsc.VectorSubcoreMesh(core_axis_name="c", subcore_axis_name="s")


@functools.partial(
    pl.kernel,
    mesh=_mesh,
    out_type=jax.ShapeDtypeStruct((_BATCH, _EMBEDDING_DIM), jnp.float32),
    scratch_types=[
        pltpu.VMEM((_B_PER_W,), jnp.int32),
        pltpu.VMEM((_B_PER_W, _EMBEDDING_DIM), jnp.float32),
        pltpu.SemaphoreType.DMA,
    ],
    compiler_params=pltpu.CompilerParams(
        needs_layout_passes=False,
        skip_device_barrier=True,
        disable_bounds_checks=True,
        disable_semaphore_checks=True,
    ),
)
def _gather_rows(table_hbm, idx_hbm, out_hbm, idx_v, rows_v, sem):
    wid = lax.axis_index("s") * _NUM_CORES + lax.axis_index("c")
    base = wid * _B_PER_W
    pltpu.sync_copy(idx_hbm.at[pl.ds(base, _B_PER_W)], idx_v)

    lane_iota = lax.iota(jnp.int32, _LANES)

    def _issue(i, carry):
        chunk = idx_v[pl.ds((i // _LANES) * _LANES, _LANES)]
        v = jnp.where(lane_iota == (i % _LANES), chunk, 0)
        r = jnp.max(v)
        pltpu.make_async_copy(
            table_hbm.at[pl.ds(r, 1)], rows_v.at[pl.ds(i, 1)], sem
        ).start()
        return carry

    lax.fori_loop(0, _B_PER_W, _issue, 0)

    pltpu.make_async_copy(
        table_hbm.at[pl.ds(0, _B_PER_W)], rows_v, sem
    ).wait()

    pltpu.sync_copy(rows_v, out_hbm.at[pl.ds(base, _B_PER_W)])


def kernel(entities, entity_embeddings):
    return _gather_rows(entity_embeddings, entities.astype(jnp.int32))


# speedup vs baseline: 1.7494x; 1.7494x over previous
import functools

import jax
import jax.numpy as jnp
from jax import lax
from jax.experimental import pallas as pl
from jax.experimental.pallas import tpu as pltpu
from jax.experimental.pallas import tpu_sc as plsc

_NUM_ENTITIES = 100000
_EMBEDDING_DIM = 64
_BATCH = 4096

_NUM_CORES = 2
_NUM_SUBCORES = 16
_NUM_WORKERS = _NUM_CORES * _NUM_SUBCORES
_COLS_PER_W = _EMBEDDING_DIM // _NUM_WORKERS
_LANES = 16
_CHUNKS = _BATCH // _LANES

_mesh = plsc.VectorSubcoreMesh(core_axis_name="c", subcore_axis_name="s")


@functools.partial(
    pl.kernel,
    mesh=_mesh,
    out_type=jax.ShapeDtypeStruct((_EMBEDDING_DIM, _BATCH), jnp.float32),
    scratch_types=[
        pltpu.VMEM((_BATCH,), jnp.int32),
        pltpu.VMEM((_NUM_ENTITIES,), jnp.float32),
        pltpu.VMEM((_BATCH,), jnp.float32),
        pltpu.SemaphoreType.DMA,
    ],
    compiler_params=pltpu.CompilerParams(
        needs_layout_passes=False,
        disable_bounds_checks=True,
        disable_semaphore_checks=True,
    ),
)
def _gather_cols(table_t_hbm, idx_hbm, out_t_hbm, idx_v, col_v, outcol_v, sem):
    wid = lax.axis_index("s") * _NUM_CORES + lax.axis_index("c")
    pltpu.sync_copy(idx_hbm, idx_v)

    for j in range(_COLS_PER_W):
        c = wid * _COLS_PER_W + j
        pltpu.sync_copy(table_t_hbm.at[c], col_v)

        def _gather(k, carry):
            ii = idx_v[pl.ds(k * _LANES, _LANES)]
            outcol_v[pl.ds(k * _LANES, _LANES)] = plsc.load_gather(
                col_v, [ii]
            )
            return carry

        lax.fori_loop(0, _CHUNKS, _gather, 0)

        pltpu.sync_copy(outcol_v, out_t_hbm.at[c])


def kernel(entities, entity_embeddings):
    out_t = _gather_cols(entity_embeddings.T, entities.astype(jnp.int32))
    return out_t.T


# speedup vs baseline: 1.7734x; 1.0137x over previous
import functools

import jax
import jax.numpy as jnp
from jax import lax
from jax.experimental import pallas as pl
from jax.experimental.pallas import tpu as pltpu
from jax.experimental.pallas import tpu_sc as plsc

_NUM_ENTITIES = 100000
_EMBEDDING_DIM = 64
_BATCH = 4096

_NUM_CORES = 2
_NUM_SUBCORES = 16
_NUM_WORKERS = _NUM_CORES * _NUM_SUBCORES
_COLS_PER_W = _EMBEDDING_DIM // _NUM_WORKERS
_LANES = 16
_CHUNKS = _BATCH // _LANES

_mesh = plsc.VectorSubcoreMesh(core_axis_name="c", subcore_axis_name="s")


@functools.partial(
    pl.kernel,
    mesh=_mesh,
    out_type=jax.ShapeDtypeStruct((_EMBEDDING_DIM, _BATCH), jnp.float32),
    scratch_types=[
        pltpu.VMEM((_BATCH,), jnp.int32),
        pltpu.VMEM((_NUM_ENTITIES,), jnp.float32),
        pltpu.VMEM((_BATCH,), jnp.float32),
        pltpu.SemaphoreType.DMA,
    ],
    compiler_params=pltpu.CompilerParams(
        needs_layout_passes=False,
        disable_bounds_checks=True,
        disable_semaphore_checks=True,
    ),
)
def _gather_cols(table_t_hbm, idx_hbm, out_t_hbm, idx_v, col_v, outcol_v, sem):
    wid = lax.axis_index("s") * _NUM_CORES + lax.axis_index("c")
    pltpu.sync_copy(idx_hbm, idx_v)

    for j in range(_COLS_PER_W):
        c = wid * _COLS_PER_W + j
        pltpu.sync_copy(table_t_hbm.at[c], col_v)

        def _gather(k, carry):
            for q in range(4):
                off = k * (4 * _LANES) + q * _LANES
                ii = idx_v[pl.ds(off, _LANES)]
                outcol_v[pl.ds(off, _LANES)] = plsc.load_gather(
                    col_v, [ii]
                )
            return carry

        lax.fori_loop(0, _CHUNKS // 4, _gather, 0)

        pltpu.sync_copy(outcol_v, out_t_hbm.at[c])


def kernel(entities, entity_embeddings):
    out_t = _gather_cols(entity_embeddings.T, entities.astype(jnp.int32))
    return out_t.T


# speedup vs baseline: 1.8274x; 1.0304x over previous
import functools

import jax
import jax.numpy as jnp
from jax import lax
from jax.experimental import pallas as pl
from jax.experimental.pallas import tpu as pltpu
from jax.experimental.pallas import tpu_sc as plsc

_NUM_ENTITIES = 100000
_EMBEDDING_DIM = 64
_BATCH = 4096

_NUM_CORES = 2
_NUM_SUBCORES = 16
_NUM_WORKERS = _NUM_CORES * _NUM_SUBCORES
_COLS_PER_W = _EMBEDDING_DIM // _NUM_WORKERS
_LANES = 16
_CHUNKS = _BATCH // _LANES

_mesh = plsc.VectorSubcoreMesh(core_axis_name="c", subcore_axis_name="s")


@functools.partial(
    pl.kernel,
    mesh=_mesh,
    out_type=jax.ShapeDtypeStruct((_EMBEDDING_DIM, _BATCH), jnp.float32),
    scratch_types=[
        pltpu.VMEM((_BATCH,), jnp.int32),
        pltpu.VMEM((_NUM_ENTITIES,), jnp.float32),
        pltpu.VMEM((_BATCH,), jnp.float32),
        pltpu.VMEM((_BATCH,), jnp.float32),
        pltpu.SemaphoreType.DMA,
        pltpu.SemaphoreType.DMA,
    ],
    compiler_params=pltpu.CompilerParams(
        needs_layout_passes=False,
        disable_bounds_checks=True,
        disable_semaphore_checks=True,
    ),
)
def _gather_cols(table_t_hbm, idx_hbm, out_t_hbm, idx_v, col_v, out0_v,
                 out1_v, sem, sem_w):
    wid = lax.axis_index("s") * _NUM_CORES + lax.axis_index("c")
    c0 = wid * _COLS_PER_W

    col_copy = pltpu.make_async_copy(table_t_hbm.at[c0], col_v, sem)
    col_copy.start()
    pltpu.sync_copy(idx_hbm, idx_v)

    def _gather_into(out_v):
        def _body(k, carry):
            for q in range(4):
                off = k * (4 * _LANES) + q * _LANES
                ii = idx_v[pl.ds(off, _LANES)]
                out_v[pl.ds(off, _LANES)] = plsc.load_gather(col_v, [ii])
            return carry

        lax.fori_loop(0, _CHUNKS // 4, _body, 0)

    col_copy.wait()
    _gather_into(out0_v)
    pltpu.make_async_copy(table_t_hbm.at[c0 + 1], col_v, sem).start()
    out0_copy = pltpu.make_async_copy(out0_v, out_t_hbm.at[c0], sem_w)
    out0_copy.start()

    pltpu.make_async_copy(table_t_hbm.at[c0 + 1], col_v, sem).wait()
    _gather_into(out1_v)
    out0_copy.wait()
    pltpu.sync_copy(out1_v, out_t_hbm.at[c0 + 1])


def kernel(entities, entity_embeddings):
    out_t = _gather_cols(entity_embeddings.T, entities.astype(jnp.int32))
    return out_t.T
